# baseline (speedup 1.0000x reference)
import functools

import jax
import jax.numpy as jnp
from jax import lax
from jax.experimental import pallas as pl
from jax.experimental.pallas import tpu as pltpu

_NBUF = 8
_CHS = 64


def _body(td_ref, mask_ref, lae_ref, ltm_ref, e2_ref, ll_hbm,
          ev_ref, ne_ref, cnt_ref,
          bufs, acc_ne, acc_ev, acc_cnt, sems, *, inv_n, k):
    i = pl.program_id(0)
    nsteps = pl.num_programs(0)
    chs = _CHS

    def _copy(j, slot):
        return pltpu.make_async_copy(
            ll_hbm.at[:, pl.ds(j * chs, chs), :], bufs.at[slot], sems.at[slot])

    @pl.when(i == 0)
    def _init():
        acc_ne[...] = jnp.zeros_like(acc_ne)
        acc_ev[...] = jnp.zeros_like(acc_ev)
        acc_cnt[...] = jnp.zeros_like(acc_cnt)
        for b in range(_NBUF):
            _copy(b, b).start()

    for b in range(_NBUF):
        j = i * _NBUF + b
        _copy(j, b).wait()
        mch = mask_ref[:, b * chs:(b + 1) * chs]
        w = td_ref[:, b * chs:(b + 1) * chs] * mch * inv_n
        acc_ne[...] += lax.dot_general(
            w, bufs[b],
            dimension_numbers=(((1,), (1,)), ((0,), (0,))),
            preferred_element_type=jnp.float32,
        )

        x = (lae_ref[:, b * chs * k:(b + 1) * chs * k]
             * ltm_ref[:, b * chs * k:(b + 1) * chs * k])
        ev_l = jnp.dot(x, e2_ref[...], preferred_element_type=jnp.float32)
        acc_ev[...] += jnp.log(jnp.where(mch > 0, ev_l, 1.0))
        acc_cnt[...] += mch

        @pl.when(i + 1 < nsteps)
        def _next():
            _copy((i + 1) * _NBUF + b, b).start()

    @pl.when(i == nsteps - 1)
    def _fini():
        ne_ref[0, 0] = jnp.sum(acc_ne[...])
        ev_ref[0, 0] = jnp.sum(acc_ev[...])
        cnt_ref[0, 0] = jnp.sum(acc_cnt[...]).astype(jnp.int32)


def kernel(time_delta_seq, lambda_at_event, lambdas_loss_samples, seq_mask, lambda_type_mask):
    B, S, N, K = lambdas_loss_samples.shape
    NK = N * K
    ll = lambdas_loss_samples.reshape(B, S, NK)
    laef = lambda_at_event.reshape(B, S * K)
    ltmf = lambda_type_mask.reshape(B, S * K)
    maskf = seq_mask.astype(jnp.float32)
    e2 = jnp.kron(jnp.eye(_CHS, dtype=jnp.float32), jnp.ones((K, 1), jnp.float32))

    nchunk = S // _CHS
    grid = (nchunk // _NBUF,)
    span = _CHS * _NBUF

    body = functools.partial(_body, inv_n=1.0 / N, k=K)
    ev, ne, cnt = pl.pallas_call(
        body,
        grid=grid,
        in_specs=[
            pl.BlockSpec((B, span), lambda i: (0, i)),
            pl.BlockSpec((B, span), lambda i: (0, i)),
            pl.BlockSpec((B, span * K), lambda i: (0, i)),
            pl.BlockSpec((B, span * K), lambda i: (0, i)),
            pl.BlockSpec((_CHS * K, _CHS), lambda i: (0, 0)),
            pl.BlockSpec(memory_space=pl.ANY),
        ],
        out_specs=[
            pl.BlockSpec(memory_space=pltpu.SMEM),
            pl.BlockSpec(memory_space=pltpu.SMEM),
            pl.BlockSpec(memory_space=pltpu.SMEM),
        ],
        out_shape=[
            jax.ShapeDtypeStruct((1, 1), jnp.float32),
            jax.ShapeDtypeStruct((1, 1), jnp.float32),
            jax.ShapeDtypeStruct((1, 1), jnp.int32),
        ],
        scratch_shapes=[
            pltpu.VMEM((_NBUF, B, _CHS, NK), jnp.float32),
            pltpu.VMEM((B, NK), jnp.float32),
            pltpu.VMEM((B, _CHS), jnp.float32),
            pltpu.VMEM((B, _CHS), jnp.float32),
            pltpu.SemaphoreType.DMA((_NBUF,)),
        ],
    )(time_delta_seq, maskf, laef, ltmf, e2, ll)

    return (ev[0, 0], ne[0, 0], cnt[0, 0])
